# Initial kernel scaffold; baseline (speedup 1.0000x reference)
import jax
import jax.numpy as jnp
from jax.experimental import pallas as pl


def kernel(fiedler_values, current_idx, goal_idx, neighbor_indices):
    raise NotImplementedError("write your pallas kernel here")


# speedup vs baseline: 231.0108x; 231.0108x over previous
import functools

import jax
import jax.numpy as jnp
from jax import lax
from jax.experimental import pallas as pl
from jax.experimental.pallas import tpu as pltpu
from jax.experimental.pallas import tpu_sc as plsc

M_NODES = 100000
K_NEIGH = 6400000
NC = 2
NS = 16
NW = NC * NS
L = 16
PER_W = K_NEIGH // NW
CHUNK = 10000
N_CHUNKS = PER_W // CHUNK
VECS = CHUNK // L

_mesh = plsc.VectorSubcoreMesh(core_axis_name="c", subcore_axis_name="s")
_params = pltpu.CompilerParams(needs_layout_passes=False)


def _wid():
    return lax.axis_index("s") * NC + lax.axis_index("c")


@functools.partial(
    pl.kernel,
    mesh=_mesh,
    out_type=jax.ShapeDtypeStruct((NW * 2 * L,), jnp.float32),
    compiler_params=_params,
    scratch_types=[
        pltpu.VMEM((M_NODES,), jnp.float32),
        pltpu.VMEM((CHUNK,), jnp.int32),
        pltpu.VMEM((2 * L,), jnp.float32),
    ],
)
def _minmax_kernel(fied_hbm, idx_hbm, out_hbm, table_v, idx_v, mm_v):
    wid = _wid()
    base = wid * PER_W
    pltpu.sync_copy(fied_hbm, table_v)
    inf = jnp.full((L,), jnp.inf, dtype=jnp.float32)

    def chunk_body(c, carry):
        pltpu.sync_copy(idx_hbm.at[pl.ds(base + c * CHUNK, CHUNK)], idx_v)

        def vec_body(i, carry2):
            vmin, vmax = carry2
            iv = idx_v[pl.ds(i * L, L)]
            g = plsc.load_gather(table_v, [iv])
            return jnp.minimum(vmin, g), jnp.maximum(vmax, g)

        return lax.fori_loop(0, VECS, vec_body, carry)

    vmin, vmax = lax.fori_loop(0, N_CHUNKS, chunk_body, (inf, -inf))
    mm_v[pl.ds(0, L)] = vmin
    mm_v[pl.ds(L, L)] = vmax
    pltpu.sync_copy(mm_v, out_hbm.at[pl.ds(wid * 2 * L, 2 * L)])


@functools.partial(
    pl.kernel,
    mesh=_mesh,
    out_type=jax.ShapeDtypeStruct((K_NEIGH,), jnp.float32),
    compiler_params=_params,
    scratch_types=[
        pltpu.VMEM((M_NODES,), jnp.float32),
        pltpu.VMEM((CHUNK,), jnp.int32),
        pltpu.VMEM((CHUNK,), jnp.float32),
        pltpu.VMEM((2 * L,), jnp.float32),
    ],
)
def _emit_kernel(fied_hbm, idx_hbm, ab_hbm, out_hbm, table_v, idx_v, out_v, ab_v):
    wid = _wid()
    base = wid * PER_W
    pltpu.sync_copy(fied_hbm, table_v)
    pltpu.sync_copy(ab_hbm, ab_v)
    a = ab_v[pl.ds(0, L)]
    b = ab_v[pl.ds(L, L)]

    def chunk_body(c, _):
        off = base + c * CHUNK
        pltpu.sync_copy(idx_hbm.at[pl.ds(off, CHUNK)], idx_v)

        def vec_body(i, _):
            iv = idx_v[pl.ds(i * L, L)]
            g = plsc.load_gather(table_v, [iv])
            out_v[pl.ds(i * L, L)] = g * a + b
            return 0

        lax.fori_loop(0, VECS, vec_body, 0)
        pltpu.sync_copy(out_v, out_hbm.at[pl.ds(off, CHUNK)])
        return 0

    lax.fori_loop(0, N_CHUNKS, chunk_body, 0)


def kernel(fiedler_values, current_idx, goal_idx, neighbor_indices):
    f32 = jnp.float32
    idx = neighbor_indices.astype(jnp.int32)

    mm = _minmax_kernel(fiedler_values, idx).reshape(NW, 2 * L)
    vmin = jnp.min(mm[:, :L])
    vmax = jnp.max(mm[:, L:])

    cur = fiedler_values[current_idx]
    goal_nonneg = goal_idx >= 0
    safe_goal = jnp.where(goal_nonneg, goal_idx, 0)
    goal_val = jnp.where(goal_nonneg, fiedler_values[safe_goal], f32(0.0))
    draw = goal_val - cur
    d = jnp.sign(draw)
    d = jnp.where(jnp.abs(draw) < 1e-08, jnp.ones_like(d), d)

    raw_min = jnp.where(d > 0, vmin - cur, cur - vmax)
    raw_max = jnp.where(d > 0, vmax - cur, cur - vmin)
    rng = raw_max - raw_min
    rng = jnp.where(rng > 1e-10, rng, jnp.ones_like(rng))

    a = (0.3 * d / rng).astype(f32)
    b = (0.3 * (-d * cur - raw_min) / rng).astype(f32)
    ab = jnp.concatenate([jnp.full((L,), a, f32), jnp.full((L,), b, f32)])

    return _emit_kernel(fiedler_values, idx, ab)


# speedup vs baseline: 288.6070x; 1.2493x over previous
import functools

import jax
import jax.numpy as jnp
from jax import lax
from jax.experimental import pallas as pl
from jax.experimental.pallas import tpu as pltpu
from jax.experimental.pallas import tpu_sc as plsc

M_NODES = 100000
K_NEIGH = 6400000
NC = 2
NS = 16
NW = NC * NS
L = 16
PER_W = K_NEIGH // NW

CH_A = 10000
NCH_A = PER_W // CH_A
U_A = 5
IT_A = CH_A // (L * U_A)

CH_B = 4000
NCH_B = PER_W // CH_B
U_B = 10
IT_B = CH_B // (L * U_B)

_mesh = plsc.VectorSubcoreMesh(core_axis_name="c", subcore_axis_name="s")
_params = pltpu.CompilerParams(needs_layout_passes=False)


def _wid():
    return lax.axis_index("s") * NC + lax.axis_index("c")


@functools.partial(
    pl.kernel,
    mesh=_mesh,
    out_type=jax.ShapeDtypeStruct((NW * 2 * L,), jnp.float32),
    compiler_params=_params,
    scratch_types=[
        pltpu.VMEM((M_NODES,), jnp.float32),
        pltpu.VMEM((CH_A,), jnp.int32),
        pltpu.VMEM((CH_A,), jnp.int32),
        pltpu.VMEM((2 * L,), jnp.float32),
        pltpu.SemaphoreType.DMA,
        pltpu.SemaphoreType.DMA,
        pltpu.SemaphoreType.DMA,
    ],
)
def _minmax_kernel(fied_hbm, idx_hbm, out_hbm, table_v, ib0, ib1, mm_v,
                   sem_t, si0, si1):
    wid = _wid()
    base = wid * PER_W
    pltpu.async_copy(fied_hbm, table_v, sem_t)
    pltpu.async_copy(idx_hbm.at[pl.ds(base, CH_A)], ib0, si0)
    pltpu.async_copy(idx_hbm.at[pl.ds(base + CH_A, CH_A)], ib1, si1)
    pltpu.make_async_copy(fied_hbm, table_v, sem_t).wait()

    def scan_chunk(buf, carry):
        def it(i, carry2):
            vmin, vmax = carry2
            for u in range(U_A):
                iv = buf[pl.ds((i * U_A + u) * L, L)]
                g = plsc.load_gather(table_v, [iv])
                vmin = jnp.minimum(vmin, g)
                vmax = jnp.maximum(vmax, g)
            return vmin, vmax

        return lax.fori_loop(0, IT_A, it, carry)

    def pair(i, carry):
        c0 = 2 * i
        pltpu.make_async_copy(idx_hbm.at[pl.ds(0, CH_A)], ib0, si0).wait()
        carry = scan_chunk(ib0, carry)

        @pl.when(c0 + 2 < NCH_A)
        def _():
            pltpu.async_copy(
                idx_hbm.at[pl.ds(base + (c0 + 2) * CH_A, CH_A)], ib0, si0)

        pltpu.make_async_copy(idx_hbm.at[pl.ds(0, CH_A)], ib1, si1).wait()
        carry = scan_chunk(ib1, carry)

        @pl.when(c0 + 3 < NCH_A)
        def _():
            pltpu.async_copy(
                idx_hbm.at[pl.ds(base + (c0 + 3) * CH_A, CH_A)], ib1, si1)

        return carry

    inf = jnp.full((L,), jnp.inf, dtype=jnp.float32)
    vmin, vmax = lax.fori_loop(0, NCH_A // 2, pair, (inf, -inf))
    mm_v[pl.ds(0, L)] = vmin
    mm_v[pl.ds(L, L)] = vmax
    pltpu.sync_copy(mm_v, out_hbm.at[pl.ds(wid * 2 * L, 2 * L)])


@functools.partial(
    pl.kernel,
    mesh=_mesh,
    out_type=jax.ShapeDtypeStruct((K_NEIGH,), jnp.float32),
    compiler_params=_params,
    scratch_types=[
        pltpu.VMEM((M_NODES,), jnp.float32),
        pltpu.VMEM((CH_B,), jnp.int32),
        pltpu.VMEM((CH_B,), jnp.int32),
        pltpu.VMEM((CH_B,), jnp.float32),
        pltpu.VMEM((CH_B,), jnp.float32),
        pltpu.VMEM((2 * L,), jnp.float32),
        pltpu.SemaphoreType.DMA,
        pltpu.SemaphoreType.DMA,
        pltpu.SemaphoreType.DMA,
        pltpu.SemaphoreType.DMA,
        pltpu.SemaphoreType.DMA,
    ],
)
def _emit_kernel(fied_hbm, idx_hbm, ab_hbm, out_hbm,
                 table_v, ib0, ib1, ob0, ob1, ab_v,
                 sem_t, si0, si1, so0, so1):
    wid = _wid()
    base = wid * PER_W
    pltpu.async_copy(fied_hbm, table_v, sem_t)
    pltpu.async_copy(idx_hbm.at[pl.ds(base, CH_B)], ib0, si0)
    pltpu.async_copy(idx_hbm.at[pl.ds(base + CH_B, CH_B)], ib1, si1)
    pltpu.sync_copy(ab_hbm, ab_v)
    a = ab_v[pl.ds(0, L)]
    b = ab_v[pl.ds(L, L)]
    pltpu.make_async_copy(fied_hbm, table_v, sem_t).wait()

    def compute_chunk(ib, ob):
        def it(i, _):
            for u in range(U_B):
                o = (i * U_B + u) * L
                iv = ib[pl.ds(o, L)]
                g = plsc.load_gather(table_v, [iv])
                ob[pl.ds(o, L)] = g * a + b
            return 0

        lax.fori_loop(0, IT_B, it, 0)

    def pair(i, _):
        c0 = 2 * i

        pltpu.make_async_copy(idx_hbm.at[pl.ds(0, CH_B)], ib0, si0).wait()

        @pl.when(i > 0)
        def _():
            pltpu.make_async_copy(ob0, out_hbm.at[pl.ds(0, CH_B)], so0).wait()

        compute_chunk(ib0, ob0)
        pltpu.async_copy(ob0, out_hbm.at[pl.ds(base + c0 * CH_B, CH_B)], so0)

        @pl.when(c0 + 2 < NCH_B)
        def _():
            pltpu.async_copy(
                idx_hbm.at[pl.ds(base + (c0 + 2) * CH_B, CH_B)], ib0, si0)

        pltpu.make_async_copy(idx_hbm.at[pl.ds(0, CH_B)], ib1, si1).wait()

        @pl.when(i > 0)
        def _():
            pltpu.make_async_copy(ob1, out_hbm.at[pl.ds(0, CH_B)], so1).wait()

        compute_chunk(ib1, ob1)
        pltpu.async_copy(
            ob1, out_hbm.at[pl.ds(base + (c0 + 1) * CH_B, CH_B)], so1)

        @pl.when(c0 + 3 < NCH_B)
        def _():
            pltpu.async_copy(
                idx_hbm.at[pl.ds(base + (c0 + 3) * CH_B, CH_B)], ib1, si1)

        return 0

    lax.fori_loop(0, NCH_B // 2, pair, 0)
    pltpu.make_async_copy(ob0, out_hbm.at[pl.ds(0, CH_B)], so0).wait()
    pltpu.make_async_copy(ob1, out_hbm.at[pl.ds(0, CH_B)], so1).wait()


def kernel(fiedler_values, current_idx, goal_idx, neighbor_indices):
    f32 = jnp.float32
    idx = neighbor_indices.astype(jnp.int32)

    mm = _minmax_kernel(fiedler_values, idx).reshape(NW, 2 * L)
    vmin = jnp.min(mm[:, :L])
    vmax = jnp.max(mm[:, L:])

    cur = fiedler_values[current_idx]
    goal_nonneg = goal_idx >= 0
    safe_goal = jnp.where(goal_nonneg, goal_idx, 0)
    goal_val = jnp.where(goal_nonneg, fiedler_values[safe_goal], f32(0.0))
    draw = goal_val - cur
    d = jnp.sign(draw)
    d = jnp.where(jnp.abs(draw) < 1e-08, jnp.ones_like(d), d)

    raw_min = jnp.where(d > 0, vmin - cur, cur - vmax)
    raw_max = jnp.where(d > 0, vmax - cur, cur - vmin)
    rng = raw_max - raw_min
    rng = jnp.where(rng > 1e-10, rng, jnp.ones_like(rng))

    a = (0.3 * d / rng).astype(f32)
    b = (0.3 * (-d * cur - raw_min) / rng).astype(f32)
    ab = jnp.concatenate([jnp.full((L,), a, f32), jnp.full((L,), b, f32)])

    return _emit_kernel(fiedler_values, idx, ab)


# speedup vs baseline: 366.1363x; 1.2686x over previous
import functools

import jax
import jax.numpy as jnp
from jax import lax
from jax.experimental import pallas as pl
from jax.experimental.pallas import tpu as pltpu
from jax.experimental.pallas import tpu_sc as plsc

M_NODES = 100000
K_NEIGH = 6400000
NC = 2
NS = 16
NW = NC * NS
L = 16
PER_W = K_NEIGH // NW

CH_A = 10000
NCH_A = PER_W // CH_A
U_A = 5
IT_A = CH_A // (L * U_A)

CH_B = 8000
NCH_B = PER_W // CH_B
U_B = 5
IT_B = CH_B // (L * U_B)
NG_B = NCH_B // 3

_mesh = plsc.VectorSubcoreMesh(core_axis_name="c", subcore_axis_name="s")
_params = pltpu.CompilerParams(needs_layout_passes=False)


def _wid():
    return lax.axis_index("s") * NC + lax.axis_index("c")


@functools.partial(
    pl.kernel,
    mesh=_mesh,
    out_type=jax.ShapeDtypeStruct((NW * 2 * L,), jnp.float32),
    compiler_params=_params,
    scratch_types=[
        pltpu.VMEM((M_NODES,), jnp.float32),
        pltpu.VMEM((CH_A,), jnp.int32),
        pltpu.VMEM((CH_A,), jnp.int32),
        pltpu.VMEM((2 * L,), jnp.float32),
        pltpu.SemaphoreType.DMA,
        pltpu.SemaphoreType.DMA,
        pltpu.SemaphoreType.DMA,
    ],
)
def _minmax_kernel(fied_hbm, idx_hbm, out_hbm, table_v, ib0, ib1, mm_v,
                   sem_t, si0, si1):
    wid = _wid()
    base = wid * PER_W
    pltpu.async_copy(fied_hbm, table_v, sem_t)
    pltpu.async_copy(idx_hbm.at[pl.ds(base, CH_A)], ib0, si0)
    pltpu.async_copy(idx_hbm.at[pl.ds(base + CH_A, CH_A)], ib1, si1)
    pltpu.make_async_copy(fied_hbm, table_v, sem_t).wait()

    def scan_chunk(buf, carry):
        def it(i, carry2):
            vmin, vmax = carry2
            for u in range(U_A):
                iv = buf[pl.ds((i * U_A + u) * L, L)]
                g = plsc.load_gather(table_v, [iv])
                vmin = jnp.minimum(vmin, g)
                vmax = jnp.maximum(vmax, g)
            return vmin, vmax

        return lax.fori_loop(0, IT_A, it, carry)

    def pair(i, carry):
        c0 = 2 * i
        pltpu.make_async_copy(idx_hbm.at[pl.ds(0, CH_A)], ib0, si0).wait()
        carry = scan_chunk(ib0, carry)

        @pl.when(c0 + 2 < NCH_A)
        def _():
            pltpu.async_copy(
                idx_hbm.at[pl.ds(base + (c0 + 2) * CH_A, CH_A)], ib0, si0)

        pltpu.make_async_copy(idx_hbm.at[pl.ds(0, CH_A)], ib1, si1).wait()
        carry = scan_chunk(ib1, carry)

        @pl.when(c0 + 3 < NCH_A)
        def _():
            pltpu.async_copy(
                idx_hbm.at[pl.ds(base + (c0 + 3) * CH_A, CH_A)], ib1, si1)

        return carry

    inf = jnp.full((L,), jnp.inf, dtype=jnp.float32)
    vmin, vmax = lax.fori_loop(0, NCH_A // 2, pair, (inf, -inf))
    mm_v[pl.ds(0, L)] = vmin
    mm_v[pl.ds(L, L)] = vmax
    pltpu.sync_copy(mm_v, out_hbm.at[pl.ds(wid * 2 * L, 2 * L)])


@functools.partial(
    pl.kernel,
    mesh=_mesh,
    out_type=jax.ShapeDtypeStruct((K_NEIGH,), jnp.float32),
    compiler_params=_params,
    scratch_types=[
        pltpu.VMEM((M_NODES,), jnp.float32),
        pltpu.VMEM((CH_B,), jnp.float32),
        pltpu.VMEM((CH_B,), jnp.float32),
        pltpu.VMEM((CH_B,), jnp.float32),
        pltpu.VMEM((2 * L,), jnp.float32),
        pltpu.SemaphoreType.DMA,
        pltpu.SemaphoreType.DMA,
        pltpu.SemaphoreType.DMA,
        pltpu.SemaphoreType.DMA,
        pltpu.SemaphoreType.DMA,
        pltpu.SemaphoreType.DMA,
        pltpu.SemaphoreType.DMA,
    ],
)
def _emit_kernel(fied_hbm, idxf_hbm, ab_hbm, out_hbm,
                 table_v, b0, b1, b2, ab_v,
                 sem_t, si0, si1, si2, so0, so1, so2):
    wid = _wid()
    base = wid * PER_W
    pltpu.async_copy(fied_hbm, table_v, sem_t)
    pltpu.async_copy(idxf_hbm.at[pl.ds(base, CH_B)], b0, si0)
    pltpu.async_copy(idxf_hbm.at[pl.ds(base + CH_B, CH_B)], b1, si1)
    pltpu.sync_copy(ab_hbm, ab_v)
    a = ab_v[pl.ds(0, L)]
    b = ab_v[pl.ds(L, L)]
    pltpu.make_async_copy(fied_hbm, table_v, sem_t).wait()

    def compute_chunk(buf):
        def it(i, _):
            for u in range(U_B):
                o = (i * U_B + u) * L
                iv = plsc.bitcast(buf[pl.ds(o, L)], jnp.int32)
                g = plsc.load_gather(table_v, [iv])
                buf[pl.ds(o, L)] = g * a + b
            return 0

        lax.fori_loop(0, IT_B, it, 0)

    def wait_in(buf, sem):
        pltpu.make_async_copy(idxf_hbm.at[pl.ds(0, CH_B)], buf, sem).wait()

    def wait_out(buf, sem):
        pltpu.make_async_copy(buf, out_hbm.at[pl.ds(0, CH_B)], sem).wait()

    def start_out(buf, sem, c):
        pltpu.async_copy(buf, out_hbm.at[pl.ds(base + c * CH_B, CH_B)], sem)

    def start_in(buf, sem, c):
        pltpu.async_copy(idxf_hbm.at[pl.ds(base + c * CH_B, CH_B)], buf, sem)

    def group(g, _):
        c0 = 3 * g
        wait_in(b0, si0)
        compute_chunk(b0)
        start_out(b0, so0, c0)

        @pl.when(g > 0)
        def _():
            wait_out(b2, so2)

        start_in(b2, si2, c0 + 2)

        wait_in(b1, si1)
        compute_chunk(b1)
        start_out(b1, so1, c0 + 1)
        wait_out(b0, so0)

        @pl.when(c0 + 3 < NCH_B)
        def _():
            start_in(b0, si0, c0 + 3)

        wait_in(b2, si2)
        compute_chunk(b2)
        start_out(b2, so2, c0 + 2)
        wait_out(b1, so1)

        @pl.when(c0 + 4 < NCH_B)
        def _():
            start_in(b1, si1, c0 + 4)

        return 0

    lax.fori_loop(0, NG_B, group, 0)
    wait_in(b0, si0)
    wait_out(b2, so2)
    compute_chunk(b0)
    start_out(b0, so0, NCH_B - 1)
    pltpu.make_async_copy(b0, out_hbm.at[pl.ds(0, CH_B)], so0).wait()


def kernel(fiedler_values, current_idx, goal_idx, neighbor_indices):
    f32 = jnp.float32
    idx = neighbor_indices.astype(jnp.int32)

    mm = _minmax_kernel(fiedler_values, idx).reshape(NW, 2 * L)
    vmin = jnp.min(mm[:, :L])
    vmax = jnp.max(mm[:, L:])

    cur = fiedler_values[current_idx]
    goal_nonneg = goal_idx >= 0
    safe_goal = jnp.where(goal_nonneg, goal_idx, 0)
    goal_val = jnp.where(goal_nonneg, fiedler_values[safe_goal], f32(0.0))
    draw = goal_val - cur
    d = jnp.sign(draw)
    d = jnp.where(jnp.abs(draw) < 1e-08, jnp.ones_like(d), d)

    raw_min = jnp.where(d > 0, vmin - cur, cur - vmax)
    raw_max = jnp.where(d > 0, vmax - cur, cur - vmin)
    rng = raw_max - raw_min
    rng = jnp.where(rng > 1e-10, rng, jnp.ones_like(rng))

    a = (0.3 * d / rng).astype(f32)
    b = (0.3 * (-d * cur - raw_min) / rng).astype(f32)
    ab = jnp.concatenate([jnp.full((L,), a, f32), jnp.full((L,), b, f32)])

    idx_f = lax.bitcast_convert_type(idx, f32)
    return _emit_kernel(fiedler_values, idx_f, ab)


# speedup vs baseline: 371.2328x; 1.0139x over previous
import functools

import jax
import jax.numpy as jnp
from jax import lax
from jax.experimental import pallas as pl
from jax.experimental.pallas import tpu as pltpu
from jax.experimental.pallas import tpu_sc as plsc

M_NODES = 100000
K_NEIGH = 6400000
NC = 2
NS = 16
NW = NC * NS
L = 16
PER_W = K_NEIGH // NW

CH_A = 8000
NCH_A = PER_W // CH_A
U_A = 5
IT_A = CH_A // (L * U_A)
NG_A = NCH_A // 3

CH_B = 4000
NCH_B = PER_W // CH_B
U_B = 5
IT_B = CH_B // (L * U_B)
NB_B = 5
NG_B = NCH_B // NB_B

_mesh = plsc.VectorSubcoreMesh(core_axis_name="c", subcore_axis_name="s")
_params = pltpu.CompilerParams(needs_layout_passes=False)


def _wid():
    return lax.axis_index("s") * NC + lax.axis_index("c")


@functools.partial(
    pl.kernel,
    mesh=_mesh,
    out_type=jax.ShapeDtypeStruct((NW * 2 * L,), jnp.float32),
    compiler_params=_params,
    scratch_types=[
        pltpu.VMEM((M_NODES,), jnp.float32),
        pltpu.VMEM((CH_A,), jnp.int32),
        pltpu.VMEM((CH_A,), jnp.int32),
        pltpu.VMEM((CH_A,), jnp.int32),
        pltpu.VMEM((2 * L,), jnp.float32),
        pltpu.SemaphoreType.DMA,
        pltpu.SemaphoreType.DMA,
        pltpu.SemaphoreType.DMA,
        pltpu.SemaphoreType.DMA,
    ],
)
def _minmax_kernel(fied_hbm, idx_hbm, out_hbm, table_v, ib0, ib1, ib2, mm_v,
                   sem_t, si0, si1, si2):
    wid = _wid()
    base = wid * PER_W
    bufs = (ib0, ib1, ib2)
    sems = (si0, si1, si2)
    pltpu.async_copy(fied_hbm, table_v, sem_t)
    pltpu.async_copy(idx_hbm.at[pl.ds(base, CH_A)], ib0, si0)
    pltpu.async_copy(idx_hbm.at[pl.ds(base + CH_A, CH_A)], ib1, si1)
    pltpu.make_async_copy(fied_hbm, table_v, sem_t).wait()

    def scan_chunk(buf, carry):
        def it(i, carry2):
            vmin, vmax = carry2
            for u in range(U_A):
                iv = buf[pl.ds((i * U_A + u) * L, L)]
                g = plsc.load_gather(table_v, [iv])
                vmin = jnp.minimum(vmin, g)
                vmax = jnp.maximum(vmax, g)
            return vmin, vmax

        return lax.fori_loop(0, IT_A, it, carry)

    def chunk_step(c, k, carry):
        pltpu.make_async_copy(idx_hbm.at[pl.ds(0, CH_A)], bufs[k], sems[k]).wait()

        @pl.when(c + 2 < NCH_A)
        def _():
            pltpu.async_copy(
                idx_hbm.at[pl.ds(base + (c + 2) * CH_A, CH_A)],
                bufs[(k + 2) % 3], sems[(k + 2) % 3])

        return scan_chunk(bufs[k], carry)

    def group(g, carry):
        c0 = 3 * g
        carry = chunk_step(c0, 0, carry)
        carry = chunk_step(c0 + 1, 1, carry)
        carry = chunk_step(c0 + 2, 2, carry)
        return carry

    inf = jnp.full((L,), jnp.inf, dtype=jnp.float32)
    carry = lax.fori_loop(0, NG_A, group, (inf, -inf))
    pltpu.make_async_copy(idx_hbm.at[pl.ds(0, CH_A)], ib0, si0).wait()
    vmin, vmax = scan_chunk(ib0, carry)
    mm_v[pl.ds(0, L)] = vmin
    mm_v[pl.ds(L, L)] = vmax
    pltpu.sync_copy(mm_v, out_hbm.at[pl.ds(wid * 2 * L, 2 * L)])


@functools.partial(
    pl.kernel,
    mesh=_mesh,
    out_type=jax.ShapeDtypeStruct((K_NEIGH,), jnp.float32),
    compiler_params=_params,
    scratch_types=[
        pltpu.VMEM((M_NODES,), jnp.float32),
        pltpu.VMEM((CH_B,), jnp.float32),
        pltpu.VMEM((CH_B,), jnp.float32),
        pltpu.VMEM((CH_B,), jnp.float32),
        pltpu.VMEM((CH_B,), jnp.float32),
        pltpu.VMEM((CH_B,), jnp.float32),
        pltpu.VMEM((2 * L,), jnp.float32),
        pltpu.SemaphoreType.DMA,
        pltpu.SemaphoreType.DMA,
        pltpu.SemaphoreType.DMA,
        pltpu.SemaphoreType.DMA,
        pltpu.SemaphoreType.DMA,
        pltpu.SemaphoreType.DMA,
        pltpu.SemaphoreType.DMA,
        pltpu.SemaphoreType.DMA,
        pltpu.SemaphoreType.DMA,
        pltpu.SemaphoreType.DMA,
        pltpu.SemaphoreType.DMA,
    ],
)
def _emit_kernel(fied_hbm, idxf_hbm, ab_hbm, out_hbm,
                 table_v, b0, b1, b2, b3, b4, ab_v,
                 sem_t, si0, si1, si2, si3, si4, so0, so1, so2, so3, so4):
    wid = _wid()
    base = wid * PER_W
    bufs = (b0, b1, b2, b3, b4)
    sis = (si0, si1, si2, si3, si4)
    sos = (so0, so1, so2, so3, so4)
    pltpu.async_copy(fied_hbm, table_v, sem_t)
    for k in range(3):
        pltpu.async_copy(
            idxf_hbm.at[pl.ds(base + k * CH_B, CH_B)], bufs[k], sis[k])
    pltpu.sync_copy(ab_hbm, ab_v)
    a = ab_v[pl.ds(0, L)]
    b = ab_v[pl.ds(L, L)]
    pltpu.make_async_copy(fied_hbm, table_v, sem_t).wait()

    def compute_chunk(buf):
        def it(i, _):
            for u in range(U_B):
                o = (i * U_B + u) * L
                iv = plsc.bitcast(buf[pl.ds(o, L)], jnp.int32)
                g = plsc.load_gather(table_v, [iv])
                buf[pl.ds(o, L)] = g * a + b
            return 0

        lax.fori_loop(0, IT_B, it, 0)

    def chunk_step(c, k, g):
        kn = (k + 3) % NB_B
        pltpu.make_async_copy(idxf_hbm.at[pl.ds(0, CH_B)], bufs[k], sis[k]).wait()
        compute_chunk(bufs[k])
        pltpu.async_copy(
            bufs[k], out_hbm.at[pl.ds(base + c * CH_B, CH_B)], sos[k])

        @pl.when(c > 1)
        def _():
            pltpu.make_async_copy(
                bufs[kn], out_hbm.at[pl.ds(0, CH_B)], sos[kn]).wait()

        @pl.when(c + 3 < NCH_B)
        def _():
            pltpu.async_copy(
                idxf_hbm.at[pl.ds(base + (c + 3) * CH_B, CH_B)],
                bufs[kn], sis[kn])

        return g

    def group(g, _):
        c0 = NB_B * g
        for k in range(NB_B):
            chunk_step(c0 + k, k, g)
        return 0

    lax.fori_loop(0, NG_B, group, 0)
    pltpu.make_async_copy(b3, out_hbm.at[pl.ds(0, CH_B)], so3).wait()
    pltpu.make_async_copy(b4, out_hbm.at[pl.ds(0, CH_B)], so4).wait()


def kernel(fiedler_values, current_idx, goal_idx, neighbor_indices):
    f32 = jnp.float32
    idx = neighbor_indices.astype(jnp.int32)

    mm = _minmax_kernel(fiedler_values, idx).reshape(NW, 2 * L)
    vmin = jnp.min(mm[:, :L])
    vmax = jnp.max(mm[:, L:])

    cur = fiedler_values[current_idx]
    goal_nonneg = goal_idx >= 0
    safe_goal = jnp.where(goal_nonneg, goal_idx, 0)
    goal_val = jnp.where(goal_nonneg, fiedler_values[safe_goal], f32(0.0))
    draw = goal_val - cur
    d = jnp.sign(draw)
    d = jnp.where(jnp.abs(draw) < 1e-08, jnp.ones_like(d), d)

    raw_min = jnp.where(d > 0, vmin - cur, cur - vmax)
    raw_max = jnp.where(d > 0, vmax - cur, cur - vmin)
    rng = raw_max - raw_min
    rng = jnp.where(rng > 1e-10, rng, jnp.ones_like(rng))

    a = (0.3 * d / rng).astype(f32)
    b = (0.3 * (-d * cur - raw_min) / rng).astype(f32)
    ab = jnp.concatenate([jnp.full((L,), a, f32), jnp.full((L,), b, f32)])

    idx_f = lax.bitcast_convert_type(idx, f32)
    return _emit_kernel(fiedler_values, idx_f, ab)


# speedup vs baseline: 564.0023x; 1.5193x over previous
import functools

import jax
import jax.numpy as jnp
from jax import lax
from jax.experimental import pallas as pl
from jax.experimental.pallas import tpu as pltpu
from jax.experimental.pallas import tpu_sc as plsc

M_NODES = 100000
K_NEIGH = 6400000
NC = 2
NS = 16
NW = NC * NS
L = 16
PER_W = K_NEIGH // NW

CH_A = 8000
NCH_A = PER_W // CH_A
U_A = 5
IT_A = CH_A // (L * U_A)
NG_A = NCH_A // 3

CH_B = 4000
NCH_B = PER_W // CH_B
U_B = 5
IT_B = CH_B // (L * U_B)
NB_B = 5
NG_B = NCH_B // NB_B

_mesh = plsc.VectorSubcoreMesh(core_axis_name="c", subcore_axis_name="s")
_params = pltpu.CompilerParams(needs_layout_passes=False)


def _wid():
    return lax.axis_index("s") * NC + lax.axis_index("c")


@functools.partial(
    pl.kernel,
    mesh=_mesh,
    out_type=jax.ShapeDtypeStruct((NW * 2 * L,), jnp.float32),
    compiler_params=_params,
    scratch_types=[
        pltpu.VMEM((M_NODES,), jnp.float32),
        pltpu.VMEM((CH_A,), jnp.int32),
        pltpu.VMEM((CH_A,), jnp.int32),
        pltpu.VMEM((CH_A,), jnp.int32),
        pltpu.VMEM((2 * L,), jnp.float32),
        pltpu.SemaphoreType.DMA,
        pltpu.SemaphoreType.DMA,
        pltpu.SemaphoreType.DMA,
        pltpu.SemaphoreType.DMA,
    ],
)
def _minmax_kernel(fied_hbm, idx_hbm, out_hbm, table_v, ib0, ib1, ib2, mm_v,
                   sem_t, si0, si1, si2):
    wid = _wid()
    base = wid * PER_W
    bufs = (ib0, ib1, ib2)
    sems = (si0, si1, si2)
    pltpu.async_copy(fied_hbm, table_v, sem_t)
    pltpu.async_copy(idx_hbm.at[pl.ds(base, CH_A)], ib0, si0)
    pltpu.async_copy(idx_hbm.at[pl.ds(base + CH_A, CH_A)], ib1, si1)
    pltpu.make_async_copy(fied_hbm, table_v, sem_t).wait()

    def scan_chunk(buf, carry):
        @plsc.parallel_loop(0, CH_A // L, unroll=8, carry=carry)
        def body(i, carry2):
            vmin, vmax = carry2
            iv = buf[pl.ds(i * L, L)]
            g = plsc.load_gather(table_v, [iv])
            return jnp.minimum(vmin, g), jnp.maximum(vmax, g)

        return body

    def chunk_step(c, k, carry):
        pltpu.make_async_copy(idx_hbm.at[pl.ds(0, CH_A)], bufs[k], sems[k]).wait()

        @pl.when(c + 2 < NCH_A)
        def _():
            pltpu.async_copy(
                idx_hbm.at[pl.ds(base + (c + 2) * CH_A, CH_A)],
                bufs[(k + 2) % 3], sems[(k + 2) % 3])

        return scan_chunk(bufs[k], carry)

    def group(g, carry):
        c0 = 3 * g
        carry = chunk_step(c0, 0, carry)
        carry = chunk_step(c0 + 1, 1, carry)
        carry = chunk_step(c0 + 2, 2, carry)
        return carry

    inf = jnp.full((L,), jnp.inf, dtype=jnp.float32)
    carry = lax.fori_loop(0, NG_A, group, (inf, -inf))
    pltpu.make_async_copy(idx_hbm.at[pl.ds(0, CH_A)], ib0, si0).wait()
    vmin, vmax = scan_chunk(ib0, carry)
    mm_v[pl.ds(0, L)] = vmin
    mm_v[pl.ds(L, L)] = vmax
    pltpu.sync_copy(mm_v, out_hbm.at[pl.ds(wid * 2 * L, 2 * L)])


@functools.partial(
    pl.kernel,
    mesh=_mesh,
    out_type=jax.ShapeDtypeStruct((K_NEIGH,), jnp.float32),
    compiler_params=_params,
    scratch_types=[
        pltpu.VMEM((M_NODES,), jnp.float32),
        pltpu.VMEM((CH_B,), jnp.float32),
        pltpu.VMEM((CH_B,), jnp.float32),
        pltpu.VMEM((CH_B,), jnp.float32),
        pltpu.VMEM((CH_B,), jnp.float32),
        pltpu.VMEM((CH_B,), jnp.float32),
        pltpu.VMEM((2 * L,), jnp.float32),
        pltpu.SemaphoreType.DMA,
        pltpu.SemaphoreType.DMA,
        pltpu.SemaphoreType.DMA,
        pltpu.SemaphoreType.DMA,
        pltpu.SemaphoreType.DMA,
        pltpu.SemaphoreType.DMA,
        pltpu.SemaphoreType.DMA,
        pltpu.SemaphoreType.DMA,
        pltpu.SemaphoreType.DMA,
        pltpu.SemaphoreType.DMA,
        pltpu.SemaphoreType.DMA,
    ],
)
def _emit_kernel(fied_hbm, idxf_hbm, ab_hbm, out_hbm,
                 table_v, b0, b1, b2, b3, b4, ab_v,
                 sem_t, si0, si1, si2, si3, si4, so0, so1, so2, so3, so4):
    wid = _wid()
    base = wid * PER_W
    bufs = (b0, b1, b2, b3, b4)
    sis = (si0, si1, si2, si3, si4)
    sos = (so0, so1, so2, so3, so4)
    pltpu.async_copy(fied_hbm, table_v, sem_t)
    for k in range(3):
        pltpu.async_copy(
            idxf_hbm.at[pl.ds(base + k * CH_B, CH_B)], bufs[k], sis[k])
    pltpu.sync_copy(ab_hbm, ab_v)
    a = ab_v[pl.ds(0, L)]
    b = ab_v[pl.ds(L, L)]
    pltpu.make_async_copy(fied_hbm, table_v, sem_t).wait()

    def compute_chunk(buf):
        @plsc.parallel_loop(0, CH_B // L, unroll=8)
        def body(i):
            o = i * L
            iv = plsc.bitcast(buf[pl.ds(o, L)], jnp.int32)
            g = plsc.load_gather(table_v, [iv])
            buf[pl.ds(o, L)] = g * a + b

    def chunk_step(c, k, g):
        kn = (k + 3) % NB_B
        pltpu.make_async_copy(idxf_hbm.at[pl.ds(0, CH_B)], bufs[k], sis[k]).wait()
        compute_chunk(bufs[k])
        pltpu.async_copy(
            bufs[k], out_hbm.at[pl.ds(base + c * CH_B, CH_B)], sos[k])

        @pl.when(c > 1)
        def _():
            pltpu.make_async_copy(
                bufs[kn], out_hbm.at[pl.ds(0, CH_B)], sos[kn]).wait()

        @pl.when(c + 3 < NCH_B)
        def _():
            pltpu.async_copy(
                idxf_hbm.at[pl.ds(base + (c + 3) * CH_B, CH_B)],
                bufs[kn], sis[kn])

        return g

    def group(g, _):
        c0 = NB_B * g
        for k in range(NB_B):
            chunk_step(c0 + k, k, g)
        return 0

    lax.fori_loop(0, NG_B, group, 0)
    pltpu.make_async_copy(b3, out_hbm.at[pl.ds(0, CH_B)], so3).wait()
    pltpu.make_async_copy(b4, out_hbm.at[pl.ds(0, CH_B)], so4).wait()


def kernel(fiedler_values, current_idx, goal_idx, neighbor_indices):
    f32 = jnp.float32
    idx = neighbor_indices.astype(jnp.int32)

    mm = _minmax_kernel(fiedler_values, idx).reshape(NW, 2 * L)
    vmin = jnp.min(mm[:, :L])
    vmax = jnp.max(mm[:, L:])

    cur = fiedler_values[current_idx]
    goal_nonneg = goal_idx >= 0
    safe_goal = jnp.where(goal_nonneg, goal_idx, 0)
    goal_val = jnp.where(goal_nonneg, fiedler_values[safe_goal], f32(0.0))
    draw = goal_val - cur
    d = jnp.sign(draw)
    d = jnp.where(jnp.abs(draw) < 1e-08, jnp.ones_like(d), d)

    raw_min = jnp.where(d > 0, vmin - cur, cur - vmax)
    raw_max = jnp.where(d > 0, vmax - cur, cur - vmin)
    rng = raw_max - raw_min
    rng = jnp.where(rng > 1e-10, rng, jnp.ones_like(rng))

    a = (0.3 * d / rng).astype(f32)
    b = (0.3 * (-d * cur - raw_min) / rng).astype(f32)
    ab = jnp.concatenate([jnp.full((L,), a, f32), jnp.full((L,), b, f32)])

    idx_f = lax.bitcast_convert_type(idx, f32)
    return _emit_kernel(fiedler_values, idx_f, ab)


# speedup vs baseline: 596.0191x; 1.0568x over previous
import functools

import jax
import jax.numpy as jnp
from jax import lax
from jax.experimental import pallas as pl
from jax.experimental.pallas import tpu as pltpu
from jax.experimental.pallas import tpu_sc as plsc

M_NODES = 100000
K_NEIGH = 6400000
NC = 2
NS = 16
NW = NC * NS
L = 16
PER_W = K_NEIGH // NW

CH_A = 8000
NCH_A = PER_W // CH_A
U_A = 5
IT_A = CH_A // (L * U_A)
NG_A = NCH_A // 3

CH_B = 4000
NCH_B = PER_W // CH_B
U_B = 5
IT_B = CH_B // (L * U_B)
NB_B = 5
NG_B = NCH_B // NB_B

_mesh = plsc.VectorSubcoreMesh(core_axis_name="c", subcore_axis_name="s")
_params = pltpu.CompilerParams(needs_layout_passes=False)


def _wid():
    return lax.axis_index("s") * NC + lax.axis_index("c")


@functools.partial(
    pl.kernel,
    mesh=_mesh,
    out_type=jax.ShapeDtypeStruct((NW * 2 * L,), jnp.float32),
    compiler_params=_params,
    scratch_types=[
        pltpu.VMEM((M_NODES,), jnp.float32),
        pltpu.VMEM((CH_A,), jnp.int32),
        pltpu.VMEM((CH_A,), jnp.int32),
        pltpu.VMEM((CH_A,), jnp.int32),
        pltpu.VMEM((2 * L,), jnp.float32),
        pltpu.SemaphoreType.DMA,
        pltpu.SemaphoreType.DMA,
        pltpu.SemaphoreType.DMA,
        pltpu.SemaphoreType.DMA,
    ],
)
def _minmax_kernel(fied_hbm, idx_hbm, out_hbm, table_v, ib0, ib1, ib2, mm_v,
                   sem_t, si0, si1, si2):
    wid = _wid()
    base = wid * PER_W
    bufs = (ib0, ib1, ib2)
    sems = (si0, si1, si2)
    pltpu.async_copy(fied_hbm, table_v, sem_t)
    pltpu.async_copy(idx_hbm.at[pl.ds(base, CH_A)], ib0, si0)
    pltpu.async_copy(idx_hbm.at[pl.ds(base + CH_A, CH_A)], ib1, si1)
    pltpu.make_async_copy(fied_hbm, table_v, sem_t).wait()

    def scan_chunk(buf, carry):
        @plsc.parallel_loop(0, CH_A // (L * 4), unroll=4, carry=carry)
        def body(i, carry2):
            out = []
            for u in range(4):
                iv = buf[pl.ds((i * 4 + u) * L, L)]
                g = plsc.load_gather(table_v, [iv])
                m, x = carry2[u]
                out.append((jnp.minimum(m, g), jnp.maximum(x, g)))
            return tuple(out)

        return body

    def chunk_step(c, k, carry):
        pltpu.make_async_copy(idx_hbm.at[pl.ds(0, CH_A)], bufs[k], sems[k]).wait()

        @pl.when(c + 2 < NCH_A)
        def _():
            pltpu.async_copy(
                idx_hbm.at[pl.ds(base + (c + 2) * CH_A, CH_A)],
                bufs[(k + 2) % 3], sems[(k + 2) % 3])

        return scan_chunk(bufs[k], carry)

    def group(g, carry):
        c0 = 3 * g
        carry = chunk_step(c0, 0, carry)
        carry = chunk_step(c0 + 1, 1, carry)
        carry = chunk_step(c0 + 2, 2, carry)
        return carry

    inf = jnp.full((L,), jnp.inf, dtype=jnp.float32)
    carry0 = tuple((inf, -inf) for _ in range(4))
    carry = lax.fori_loop(0, NG_A, group, carry0)
    pltpu.make_async_copy(idx_hbm.at[pl.ds(0, CH_A)], ib0, si0).wait()
    carry = scan_chunk(ib0, carry)
    vmin = jnp.minimum(jnp.minimum(carry[0][0], carry[1][0]),
                       jnp.minimum(carry[2][0], carry[3][0]))
    vmax = jnp.maximum(jnp.maximum(carry[0][1], carry[1][1]),
                       jnp.maximum(carry[2][1], carry[3][1]))
    mm_v[pl.ds(0, L)] = vmin
    mm_v[pl.ds(L, L)] = vmax
    pltpu.sync_copy(mm_v, out_hbm.at[pl.ds(wid * 2 * L, 2 * L)])


@functools.partial(
    pl.kernel,
    mesh=_mesh,
    out_type=jax.ShapeDtypeStruct((K_NEIGH,), jnp.float32),
    compiler_params=_params,
    scratch_types=[
        pltpu.VMEM((M_NODES,), jnp.float32),
        pltpu.VMEM((CH_B,), jnp.float32),
        pltpu.VMEM((CH_B,), jnp.float32),
        pltpu.VMEM((CH_B,), jnp.float32),
        pltpu.VMEM((CH_B,), jnp.float32),
        pltpu.VMEM((CH_B,), jnp.float32),
        pltpu.VMEM((NW * 2 * L,), jnp.float32),
        pltpu.VMEM((2 * L,), jnp.int32),
        pltpu.SemaphoreType.DMA,
        pltpu.SemaphoreType.DMA,
        pltpu.SemaphoreType.DMA,
        pltpu.SemaphoreType.DMA,
        pltpu.SemaphoreType.DMA,
        pltpu.SemaphoreType.DMA,
        pltpu.SemaphoreType.DMA,
        pltpu.SemaphoreType.DMA,
        pltpu.SemaphoreType.DMA,
        pltpu.SemaphoreType.DMA,
        pltpu.SemaphoreType.DMA,
    ],
)
def _emit_kernel(fied_hbm, idxf_hbm, mm_hbm, cg_hbm, out_hbm,
                 table_v, b0, b1, b2, b3, b4, mm_v, cg_v,
                 sem_t, si0, si1, si2, si3, si4, so0, so1, so2, so3, so4):
    wid = _wid()
    base = wid * PER_W
    bufs = (b0, b1, b2, b3, b4)
    sis = (si0, si1, si2, si3, si4)
    sos = (so0, so1, so2, so3, so4)
    pltpu.async_copy(fied_hbm, table_v, sem_t)
    for k in range(3):
        pltpu.async_copy(
            idxf_hbm.at[pl.ds(base + k * CH_B, CH_B)], bufs[k], sis[k])
    pltpu.sync_copy(mm_hbm, mm_v)
    pltpu.sync_copy(cg_hbm, cg_v)
    pltpu.make_async_copy(fied_hbm, table_v, sem_t).wait()

    f32 = jnp.float32
    inf = jnp.full((L,), jnp.inf, dtype=f32)

    def red(w, carry2):
        vmin, vmax = carry2
        vmin = jnp.minimum(vmin, mm_v[pl.ds(w * 2 * L, L)])
        vmax = jnp.maximum(vmax, mm_v[pl.ds(w * 2 * L + L, L)])
        return vmin, vmax

    vmin_l, vmax_l = lax.fori_loop(0, NW, red, (inf, -inf))
    vmin = jnp.full((L,), jnp.min(vmin_l), dtype=f32)
    vmax = jnp.full((L,), jnp.max(vmax_l), dtype=f32)

    cur_i = cg_v[pl.ds(0, L)]
    goal_i = cg_v[pl.ds(L, L)]
    cur = plsc.load_gather(table_v, [cur_i])
    goal_nonneg = goal_i >= 0
    safe_goal = jnp.where(goal_nonneg, goal_i, jnp.zeros_like(goal_i))
    goal_val = jnp.where(
        goal_nonneg, plsc.load_gather(table_v, [safe_goal]),
        jnp.zeros_like(cur))
    draw = goal_val - cur
    d = jnp.sign(draw)
    d = jnp.where(jnp.abs(draw) < 1e-08, jnp.ones_like(d), d)

    raw_min = jnp.where(d > 0, vmin - cur, cur - vmax)
    raw_max = jnp.where(d > 0, vmax - cur, cur - vmin)
    rng = raw_max - raw_min
    rng = jnp.where(rng > 1e-10, rng, jnp.ones_like(rng))

    a = 0.3 * d / rng
    b = 0.3 * (-d * cur - raw_min) / rng

    def compute_chunk(buf):
        @plsc.parallel_loop(0, CH_B // L, unroll=8)
        def body(i):
            o = i * L
            iv = plsc.bitcast(buf[pl.ds(o, L)], jnp.int32)
            g = plsc.load_gather(table_v, [iv])
            buf[pl.ds(o, L)] = g * a + b

    def chunk_step(c, k, g):
        kn = (k + 3) % NB_B
        pltpu.make_async_copy(idxf_hbm.at[pl.ds(0, CH_B)], bufs[k], sis[k]).wait()
        compute_chunk(bufs[k])
        pltpu.async_copy(
            bufs[k], out_hbm.at[pl.ds(base + c * CH_B, CH_B)], sos[k])

        @pl.when(c > 1)
        def _():
            pltpu.make_async_copy(
                bufs[kn], out_hbm.at[pl.ds(0, CH_B)], sos[kn]).wait()

        @pl.when(c + 3 < NCH_B)
        def _():
            pltpu.async_copy(
                idxf_hbm.at[pl.ds(base + (c + 3) * CH_B, CH_B)],
                bufs[kn], sis[kn])

        return g

    def group(g, _):
        c0 = NB_B * g
        for k in range(NB_B):
            chunk_step(c0 + k, k, g)
        return 0

    lax.fori_loop(0, NG_B, group, 0)
    pltpu.make_async_copy(b3, out_hbm.at[pl.ds(0, CH_B)], so3).wait()
    pltpu.make_async_copy(b4, out_hbm.at[pl.ds(0, CH_B)], so4).wait()


def kernel(fiedler_values, current_idx, goal_idx, neighbor_indices):
    i32 = jnp.int32
    idx = neighbor_indices.astype(i32)

    mm = _minmax_kernel(fiedler_values, idx)

    cg = jnp.concatenate([
        jnp.full((L,), jnp.asarray(current_idx, i32)),
        jnp.full((L,), jnp.asarray(goal_idx, i32)),
    ])

    idx_f = lax.bitcast_convert_type(idx, jnp.float32)
    return _emit_kernel(fiedler_values, idx_f, mm, cg)
